# baseline (speedup 1.0000x reference)
import functools

import jax
import jax.numpy as jnp
from jax import lax
from jax.experimental import pallas as pl
from jax.experimental.pallas import tpu as pltpu
from jax.experimental.pallas import tpu_sc as plsc

C = 256
H = 128
NCORES = 2
NSUB = 16
NW = NCORES * NSUB
KCH = 128
KCH_S = 64

F32 = jnp.float32
BF16 = jnp.bfloat16


def _pad_rows(a, n):
    return jnp.pad(a, ((0, n - a.shape[0]),) + ((0, 0),) * (a.ndim - 1))


def _pad_cols(a, n):
    return jnp.pad(a, ((0, 0), (0, n - a.shape[1])))


def _pack2(a):
    bl = lax.bitcast_convert_type(a[:, :H], jnp.uint32)
    bh = lax.bitcast_convert_type(a[:, H:], jnp.uint32)
    lo = (bl + jnp.uint32(0x8000)) >> 16
    hi = (bh + jnp.uint32(0x8000)) & jnp.uint32(0xFFFF0000)
    return lax.bitcast_convert_type(hi | lo, jnp.int32)


def _unpack2(u):
    v = lax.bitcast_convert_type(u, jnp.uint32)
    lo = lax.bitcast_convert_type(v << 16, F32)
    hi = lax.bitcast_convert_type(v & jnp.uint32(0xFFFF0000), F32)
    return jnp.concatenate([lo, hi], axis=-1)


def sc_gather2(table_a, table_b, sidx, didx):
    dt = table_a.dtype
    E = sidx.shape[0]
    per_w = E // NW
    nch = per_w // KCH
    assert nch % 2 == 0
    mesh = plsc.VectorSubcoreMesh(core_axis_name="c", subcore_axis_name="s")

    @functools.partial(
        pl.kernel,
        mesh=mesh,
        out_type=(
            jax.ShapeDtypeStruct((E, H), dt),
            jax.ShapeDtypeStruct((E, H), dt),
        ),
        scratch_types=[
            pltpu.VMEM((per_w,), jnp.int32),
            pltpu.VMEM((per_w,), jnp.int32),
            pltpu.VMEM((KCH, H), dt),
            pltpu.VMEM((KCH, H), dt),
            pltpu.VMEM((KCH, H), dt),
            pltpu.VMEM((KCH, H), dt),
            pltpu.SemaphoreType.DMA,
            pltpu.SemaphoreType.DMA,
        ],
    )
    def k(ta, tb, si, di, oa, ob, si_v, di_v, a0, b0, a1, b1, sem0, sem1):
        wid = lax.axis_index("s") * NCORES + lax.axis_index("c")
        base = wid * per_w
        pltpu.sync_copy(si.at[pl.ds(base, per_w)], si_v)
        pltpu.sync_copy(di.at[pl.ds(base, per_w)], di_v)

        def fire(off, ba, bb, sem):
            pltpu.async_copy(ta.at[si_v.at[pl.ds(off, KCH)]], ba, sem)
            pltpu.async_copy(tb.at[di_v.at[pl.ds(off, KCH)]], bb, sem)

        def drain(ba, bb, sem):
            pltpu.make_async_copy(ta.at[pl.ds(0, KCH)], ba, sem).wait()
            pltpu.make_async_copy(tb.at[pl.ds(0, KCH)], bb, sem).wait()

        fire(0, a0, b0, sem0)

        def body(i, carry):
            c0 = 2 * i * KCH
            fire(c0 + KCH, a1, b1, sem1)
            drain(a0, b0, sem0)
            pltpu.sync_copy(a0, oa.at[pl.ds(base + c0, KCH)])
            pltpu.sync_copy(b0, ob.at[pl.ds(base + c0, KCH)])

            @pl.when(i + 1 < nch // 2)
            def _():
                fire(c0 + 2 * KCH, a0, b0, sem0)

            drain(a1, b1, sem1)
            pltpu.sync_copy(a1, oa.at[pl.ds(base + c0 + KCH, KCH)])
            pltpu.sync_copy(b1, ob.at[pl.ds(base + c0 + KCH, KCH)])
            return carry

        lax.fori_loop(0, nch // 2, body, 0)

    return k(table_a, table_b, sidx, didx)


def sc_scatter_add(msg2, didx, seg, rounds):
    E = didx.shape[0]
    per_t = E // NSUB
    nch = per_t // KCH_S
    assert nch % 2 == 0
    zr = seg + 128
    zr_t = zr // NSUB
    cr = seg // NSUB
    nout = rounds * seg
    mesh = plsc.VectorSubcoreMesh(core_axis_name="c", subcore_axis_name="s")

    @functools.partial(
        pl.kernel,
        mesh=mesh,
        out_type=jax.ShapeDtypeStruct((2, nout, H), F32),
        scratch_types=[
            pltpu.VMEM((per_t,), jnp.int32),
            pltpu.VMEM((KCH_S,), jnp.int32),
            pltpu.VMEM((KCH_S,), jnp.int32),
            pltpu.VMEM((KCH_S, H), F32),
            pltpu.VMEM((KCH_S, H), F32),
            pltpu.VMEM_SHARED((zr, H), F32),
            pltpu.SemaphoreType.DMA,
            pltpu.SemaphoreType.DMA,
        ],
    )
    def k(msg, di, zeros_h, out, di_v, idx0, idx1, buf0, buf1, acc, sem0, sem1):
        c = lax.axis_index("c")
        t = lax.axis_index("s")
        base = t * per_t
        pltpu.sync_copy(di.at[pl.ds(base, per_t)], di_v)

        def fire(off, buf, sem):
            pltpu.async_copy(msg.at[c, pl.ds(base + off, KCH_S)], buf, sem)

        def drain(buf, sem):
            pltpu.make_async_copy(msg.at[c, pl.ds(0, KCH_S)], buf, sem).wait()

        def xform(off, idx_v, r):
            for j in range(KCH_S // 16):
                dv = di_v[pl.ds(off + j * 16, 16)]
                lv = dv - r * seg
                lv = jnp.where((lv < 0) | (lv >= seg), seg, lv)
                idx_v[pl.ds(j * 16, 16)] = lv

        for r in range(rounds):
            pltpu.sync_copy(
                zeros_h.at[pl.ds(t * zr_t, zr_t)], acc.at[pl.ds(t * zr_t, zr_t)]
            )
            plsc.subcore_barrier()
            fire(0, buf0, sem0)

            def body(i, carry):
                c0 = 2 * i * KCH_S
                fire(c0 + KCH_S, buf1, sem1)
                xform(c0, idx0, r)
                drain(buf0, sem0)
                pltpu.sync_copy(buf0, acc.at[idx0], add=True)

                @pl.when(i + 1 < nch // 2)
                def _():
                    fire(c0 + 2 * KCH_S, buf0, sem0)

                xform(c0 + KCH_S, idx1, r)
                drain(buf1, sem1)
                pltpu.sync_copy(buf1, acc.at[idx1], add=True)
                return carry

            lax.fori_loop(0, nch // 2, body, 0)
            plsc.subcore_barrier()
            start = t * cr
            pltpu.sync_copy(
                acc.at[pl.ds(start, cr)], out.at[c, pl.ds(r * seg + start, cr)]
            )
            plsc.subcore_barrier()

    zeros_h = jnp.zeros((zr, H), F32)
    return k(msg2, didx, zeros_h)


def _dotb(a, w):
    return jnp.dot(a.astype(BF16), w.astype(BF16), preferred_element_type=F32)


def _rows_grid(n, bk):
    assert n % bk == 0
    return n // bk


def _mm_spec():
    return pl.BlockSpec((C, C), lambda i: (0, 0))


def _bias_spec():
    return pl.BlockSpec((1, C), lambda i: (0, 0))


def tc_data_node(x2, attr8, w1p, b1, w2, b2, wa_enc, wb_dec, b1_dec, v1a, c1):
    n = x2.shape[1]
    v = x2.shape[2]
    bk = 400
    grid = _rows_grid(n, bk)

    def body(x, at, w1, b1r, w2r, b2r, wa, wbd, b1d, va, c1r, o_a, o_bd, o_p):
        xcat = jnp.concatenate([x[0], x[1], at[...]], axis=-1)
        h = jax.nn.silu(_dotb(xcat, w1[...]) + b1r[...])
        xs = _dotb(h, w2r[...]) + b2r[...]
        o_a[...] = _pack2(_dotb(xs, wa[...]))
        o_bd[...] = _pack2(_dotb(xs, wbd[...]) + b1d[...])
        o_p[...] = _dotb(xs, va[...]) + c1r[...]

    row = pl.BlockSpec((bk, C), lambda i: (i, 0))
    half = pl.BlockSpec((bk, H), lambda i: (i, 0))
    return pl.pallas_call(
        body,
        grid=(grid,),
        in_specs=[pl.BlockSpec((2, bk, v), lambda i: (0, i, 0)),
                  pl.BlockSpec((bk, 8), lambda i: (i, 0)),
                  pl.BlockSpec((2 * v + 8, C), lambda i: (0, 0)),
                  _bias_spec(), _mm_spec(), _bias_spec(),
                  _mm_spec(), _mm_spec(), _bias_spec(), _mm_spec(), _bias_spec()],
        out_specs=[half, half, row],
        out_shape=[jax.ShapeDtypeStruct((n, H), jnp.int32),
                   jax.ShapeDtypeStruct((n, H), jnp.int32),
                   jax.ShapeDtypeStruct((n, C), F32)],
    )(x2, attr8, w1p, b1, w2, b2, wa_enc, wb_dec, b1_dec, v1a, c1)


def tc_hidden_node(hp, w1p, b1, w2, b2, wb_enc, b1_enc):
    n = hp.shape[0]
    bk = 400
    grid = _rows_grid(n, bk)

    def body(x, w1, b1r, w2r, b2r, wb, b1e, o_xd, o_b):
        h = jax.nn.silu(_dotb(x[...], w1[...]) + b1r[...])
        xd = _dotb(h, w2r[...]) + b2r[...]
        o_xd[...] = xd
        o_b[...] = _pack2(_dotb(xd, wb[...]) + b1e[...])

    row = pl.BlockSpec((bk, C), lambda i: (i, 0))
    in_row = pl.BlockSpec((bk, 8), lambda i: (i, 0))
    w1_spec = pl.BlockSpec((8, C), lambda i: (0, 0))
    return pl.pallas_call(
        body,
        grid=(grid,),
        in_specs=[in_row, w1_spec, _bias_spec(), _mm_spec(), _bias_spec(),
                  _mm_spec(), _bias_spec()],
        out_specs=[row, pl.BlockSpec((bk, H), lambda i: (i, 0))],
        out_shape=[jax.ShapeDtypeStruct((n, C), F32),
                   jax.ShapeDtypeStruct((10112, H), jnp.int32)],
    )(hp, w1p, b1, w2, b2, wb_enc, b1_enc)


def tc_edge_msg(ua, ub, w2, b2):
    e = ua.shape[0]
    bk = 512
    grid = _rows_grid(e, bk)

    def body(a, b, w2r, b2r, o):
        u = jax.nn.silu(_unpack2(a[...]) + _unpack2(b[...]))
        m = jnp.dot(u.astype(BF16), w2r[...],
                    preferred_element_type=F32) + b2r[...]
        o[0] = m[:, :H]
        o[1] = m[:, H:]

    half = pl.BlockSpec((bk, H), lambda i: (i, 0))
    return pl.pallas_call(
        body,
        grid=(grid,),
        in_specs=[half, half, _mm_spec(), _bias_spec()],
        out_specs=pl.BlockSpec((2, bk, H), lambda i: (0, i, 0)),
        out_shape=jax.ShapeDtypeStruct((2, e, H), F32),
    )(ua, ub, w2.astype(BF16), b2)


def _ln(y):
    m = jnp.mean(y, axis=-1, keepdims=True)
    v = jnp.mean((y - m) ** 2, axis=-1, keepdims=True)
    return (y - m) * lax.rsqrt(v + 1e-5)


def tc_ln_proj(prev, agg2, wa, wb, b1n):
    n = prev.shape[0]
    bk = 400
    grid = _rows_grid(n, bk)

    def body(p, g, war, wbr, b1r, o_xl, o_a, o_b):
        y = p[...] + jnp.concatenate([g[0], g[1]], axis=-1)
        xl = _ln(y)
        o_xl[...] = xl
        o_a[...] = _pack2(_dotb(xl, war[...]))
        o_b[...] = _pack2(_dotb(xl, wbr[...]) + b1r[...])

    row = pl.BlockSpec((bk, C), lambda i: (i, 0))
    g_spec = pl.BlockSpec((2, bk, H), lambda i: (0, i, 0))
    return pl.pallas_call(
        body,
        grid=(grid,),
        in_specs=[row, g_spec, _mm_spec(), _mm_spec(), _bias_spec()],
        out_specs=[row, pl.BlockSpec((bk, H), lambda i: (i, 0)),
                   pl.BlockSpec((bk, H), lambda i: (i, 0))],
        out_shape=[jax.ShapeDtypeStruct((n, C), F32),
                   jax.ShapeDtypeStruct((10112, H), jnp.int32),
                   jax.ShapeDtypeStruct((10112, H), jnp.int32)],
    )(prev, agg2, wa, wb, b1n)


def tc_dec_src(h1, agg2, xl, wa):
    n = h1.shape[0]
    bk = 400
    grid = _rows_grid(n, bk)

    def body(hh, g, xlr, war, o_a):
        y = hh[...] + jnp.concatenate([g[0], g[1]], axis=-1)
        o_a[...] = _pack2(_dotb(_ln(y) + xlr[...], war[...]))

    row = pl.BlockSpec((bk, C), lambda i: (i, 0))
    g_spec = pl.BlockSpec((2, bk, H), lambda i: (0, i, 0))
    return pl.pallas_call(
        body,
        grid=(grid,),
        in_specs=[row, g_spec, row, _mm_spec()],
        out_specs=pl.BlockSpec((bk, H), lambda i: (i, 0)),
        out_shape=jax.ShapeDtypeStruct((10112, H), jnp.int32),
    )(h1, agg2, xl, wa)


def tc_final(p, agg3, v1b, v2, c2, xlast):
    n = p.shape[0]
    v = v2.shape[1]
    bk = 400
    grid = _rows_grid(n, bk)

    def body(pr, g, vb, v2r, c2r, xl, o):
        gc = jnp.concatenate([g[0], g[1]], axis=-1)
        z = jax.nn.silu(pr[...] + _dotb(gc, vb[...]))
        o[...] = _dotb(z, v2r[...]) + c2r[...] + xl[...]

    row = pl.BlockSpec((bk, C), lambda i: (i, 0))
    g_spec = pl.BlockSpec((2, bk, H), lambda i: (0, i, 0))
    vrow = pl.BlockSpec((bk, v), lambda i: (i, 0))
    return pl.pallas_call(
        body,
        grid=(grid,),
        in_specs=[row, g_spec, _mm_spec(),
                  pl.BlockSpec((C, v), lambda i: (0, 0)),
                  pl.BlockSpec((1, v), lambda i: (0, 0)), vrow],
        out_specs=vrow,
        out_shape=jax.ShapeDtypeStruct((n, v), F32),
    )(p, agg3, v1b, v2, c2, xlast)


def _pad_edges(edge, n_pad, dst_pad_val):
    s = jnp.pad(edge[0], (0, n_pad - edge.shape[1]))
    d = jnp.pad(edge[1], (0, n_pad - edge.shape[1]), constant_values=dst_pad_val)
    return s, d


def kernel(x, params, edge_enc, edge_proc, edge_dec):
    b, t, e_ens, n_data, v = x.shape
    n_hidden = params["hidden_attr"].shape[0]

    x2 = x[0, :, 0]
    attr8 = _pad_cols(params["data_attr"], 8)
    hp = params["hidden_attr"]
    xlast = x[0, -1, 0]

    pe, pd = params["enc_edge"], params["dec_edge"]
    po = params["dec_out"]
    b1e = pe["b1"][None, :]
    b1d = pd["b1"][None, :]
    c1 = po["b1"][None, :]
    v2p = po["W2"]
    c2p = po["b2"][None, :]

    es, ed = params["enc_src"], params["enc_dst"]
    w1_src = _pad_rows(es["W1"], t * v + 8)
    w1_dst = ed["W1"]

    e_enc_p = 155648
    e_proc_p = 163840
    s1, d1 = _pad_edges(edge_enc, e_enc_p, n_hidden)
    s2, d2 = _pad_edges(edge_proc, e_proc_p, n_hidden)
    s3, d3 = _pad_edges(edge_dec, e_enc_p, n_data)

    a_enc, b_dec, p_out = tc_data_node(
        x2, attr8, w1_src, es["b1"][None, :], es["W2"], es["b2"][None, :],
        pe["W1"][:C], pd["W1"][C:], b1d, po["W1"][:C], c1)
    x_dst, b_enc = tc_hidden_node(
        hp, w1_dst, ed["b1"][None, :], ed["W2"], ed["b2"][None, :],
        pe["W1"][C:], b1e)

    seg_h = 10112
    ua, ub = sc_gather2(a_enc, b_enc, s1, d1)
    msg = tc_edge_msg(ua, ub, pe["W2"], pe["b2"][None, :])
    agg = sc_scatter_add(msg, d1, seg_h, 1)

    p0, p1 = params["proc"]
    xl, a0, b0 = tc_ln_proj(x_dst, agg, p0["W1"][:C], p0["W1"][C:],
                            p0["b1"][None, :])
    ua0, ub0 = sc_gather2(a0, b0, s2, d2)
    m0 = tc_edge_msg(ua0, ub0, p0["W2"], p0["b2"][None, :])
    agg0 = sc_scatter_add(m0, d2, seg_h, 1)

    h1, a1, b1 = tc_ln_proj(xl, agg0, p1["W1"][:C], p1["W1"][C:],
                            p1["b1"][None, :])
    ua1, ub1 = sc_gather2(a1, b1, s2, d2)
    m1 = tc_edge_msg(ua1, ub1, p1["W2"], p1["b2"][None, :])
    agg1 = sc_scatter_add(m1, d2, seg_h, 1)

    a_dec = tc_dec_src(h1, agg1, xl, pd["W1"][:C])
    ua3, ub3 = sc_gather2(a_dec, b_dec, s3, d3)
    m3 = tc_edge_msg(ua3, ub3, pd["W2"], pd["b2"][None, :])
    agg3 = sc_scatter_add(m3, d3, 12544, 4)

    o = tc_final(p_out, agg3, po["W1"][C:], v2p, c2p, xlast)
    return o.reshape(b, e_ens, n_data, v)


# speedup vs baseline: 1.0785x; 1.0785x over previous
import functools

import jax
import jax.numpy as jnp
from jax import lax
from jax.experimental import pallas as pl
from jax.experimental.pallas import tpu as pltpu
from jax.experimental.pallas import tpu_sc as plsc

C = 256
H = 128
NCORES = 2
NSUB = 16
NW = NCORES * NSUB
KCH = 128
KCH_S = 64

F32 = jnp.float32
BF16 = jnp.bfloat16


def _pad_rows(a, n):
    return jnp.pad(a, ((0, n - a.shape[0]),) + ((0, 0),) * (a.ndim - 1))


def _pad_cols(a, n):
    return jnp.pad(a, ((0, 0), (0, n - a.shape[1])))


def _pack2(a):
    bl = lax.bitcast_convert_type(a[:, :H], jnp.uint32)
    bh = lax.bitcast_convert_type(a[:, H:], jnp.uint32)
    lo = (bl + jnp.uint32(0x8000)) >> 16
    hi = (bh + jnp.uint32(0x8000)) & jnp.uint32(0xFFFF0000)
    return lax.bitcast_convert_type(hi | lo, jnp.int32)


def _unpack2(u):
    v = lax.bitcast_convert_type(u, jnp.uint32)
    lo = lax.bitcast_convert_type(v << 16, F32)
    hi = lax.bitcast_convert_type(v & jnp.uint32(0xFFFF0000), F32)
    return jnp.concatenate([lo, hi], axis=-1)


def sc_gather2(table_a, table_b, sidx, didx):
    dt = table_a.dtype
    E = sidx.shape[0]
    per_w = E // NW
    nch = per_w // KCH
    assert nch % 2 == 0
    mesh = plsc.VectorSubcoreMesh(core_axis_name="c", subcore_axis_name="s")

    @functools.partial(
        pl.kernel,
        mesh=mesh,
        out_type=(
            jax.ShapeDtypeStruct((E, H), dt),
            jax.ShapeDtypeStruct((E, H), dt),
        ),
        scratch_types=[
            pltpu.VMEM((per_w,), jnp.int32),
            pltpu.VMEM((per_w,), jnp.int32),
            pltpu.VMEM((KCH, H), dt),
            pltpu.VMEM((KCH, H), dt),
            pltpu.VMEM((KCH, H), dt),
            pltpu.VMEM((KCH, H), dt),
            pltpu.SemaphoreType.DMA,
            pltpu.SemaphoreType.DMA,
        ],
    )
    def k(ta, tb, si, di, oa, ob, si_v, di_v, a0, b0, a1, b1, sem0, sem1):
        wid = lax.axis_index("s") * NCORES + lax.axis_index("c")
        base = wid * per_w
        pltpu.sync_copy(si.at[pl.ds(base, per_w)], si_v)
        pltpu.sync_copy(di.at[pl.ds(base, per_w)], di_v)

        def fire(off, ba, bb, sem):
            pltpu.async_copy(ta.at[si_v.at[pl.ds(off, KCH)]], ba, sem)
            pltpu.async_copy(tb.at[di_v.at[pl.ds(off, KCH)]], bb, sem)

        def drain(ba, bb, sem):
            pltpu.make_async_copy(ta.at[pl.ds(0, KCH)], ba, sem).wait()
            pltpu.make_async_copy(tb.at[pl.ds(0, KCH)], bb, sem).wait()

        fire(0, a0, b0, sem0)

        def body(i, carry):
            c0 = 2 * i * KCH
            fire(c0 + KCH, a1, b1, sem1)
            drain(a0, b0, sem0)
            pltpu.sync_copy(a0, oa.at[pl.ds(base + c0, KCH)])
            pltpu.sync_copy(b0, ob.at[pl.ds(base + c0, KCH)])

            @pl.when(i + 1 < nch // 2)
            def _():
                fire(c0 + 2 * KCH, a0, b0, sem0)

            drain(a1, b1, sem1)
            pltpu.sync_copy(a1, oa.at[pl.ds(base + c0 + KCH, KCH)])
            pltpu.sync_copy(b1, ob.at[pl.ds(base + c0 + KCH, KCH)])
            return carry

        lax.fori_loop(0, nch // 2, body, 0)

    return k(table_a, table_b, sidx, didx)


def sc_scatter_add(msg2, didx, seg, rounds):
    E = didx.shape[0]
    per_t = E // NSUB
    nch = per_t // KCH_S
    assert nch % 2 == 0
    zr = seg + 128
    zr_t = zr // NSUB
    cr = seg // NSUB
    nout = rounds * seg
    mesh = plsc.VectorSubcoreMesh(core_axis_name="c", subcore_axis_name="s")

    @functools.partial(
        pl.kernel,
        mesh=mesh,
        out_type=jax.ShapeDtypeStruct((2, nout, H), F32),
        scratch_types=[
            pltpu.VMEM((per_t,), jnp.int32),
            pltpu.VMEM((KCH_S,), jnp.int32),
            pltpu.VMEM((KCH_S,), jnp.int32),
            pltpu.VMEM((KCH_S, H), F32),
            pltpu.VMEM((KCH_S, H), F32),
            pltpu.VMEM_SHARED((zr, H), F32),
            pltpu.SemaphoreType.DMA,
            pltpu.SemaphoreType.DMA,
        ],
    )
    def k(msg, di, zeros_h, out, di_v, idx0, idx1, buf0, buf1, acc, sem0, sem1):
        c = lax.axis_index("c")
        t = lax.axis_index("s")
        base = t * per_t
        pltpu.sync_copy(di.at[pl.ds(base, per_t)], di_v)

        def fire(off, buf, sem):
            pltpu.async_copy(msg.at[c, pl.ds(base + off, KCH_S)], buf, sem)

        def drain(buf, sem):
            pltpu.make_async_copy(msg.at[c, pl.ds(0, KCH_S)], buf, sem).wait()

        def xform(off, idx_v, r):
            for j in range(KCH_S // 16):
                dv = di_v[pl.ds(off + j * 16, 16)]
                lv = dv - r * seg
                lv = jnp.where((lv < 0) | (lv >= seg), seg, lv)
                idx_v[pl.ds(j * 16, 16)] = lv

        for r in range(rounds):
            pltpu.sync_copy(
                zeros_h.at[pl.ds(t * zr_t, zr_t)], acc.at[pl.ds(t * zr_t, zr_t)]
            )
            plsc.subcore_barrier()
            fire(0, buf0, sem0)

            def body(i, carry):
                c0 = 2 * i * KCH_S
                fire(c0 + KCH_S, buf1, sem1)
                xform(c0, idx0, r)
                drain(buf0, sem0)
                pltpu.sync_copy(buf0, acc.at[idx0], add=True)

                @pl.when(i + 1 < nch // 2)
                def _():
                    fire(c0 + 2 * KCH_S, buf0, sem0)

                xform(c0 + KCH_S, idx1, r)
                drain(buf1, sem1)
                pltpu.sync_copy(buf1, acc.at[idx1], add=True)
                return carry

            lax.fori_loop(0, nch // 2, body, 0)
            plsc.subcore_barrier()
            start = t * cr
            pltpu.sync_copy(
                acc.at[pl.ds(start, cr)], out.at[c, pl.ds(r * seg + start, cr)]
            )
            plsc.subcore_barrier()

    zeros_h = jnp.zeros((zr, H), F32)
    return k(msg2, didx, zeros_h)


def _dotb(a, w):
    return jnp.dot(a.astype(BF16), w.astype(BF16), preferred_element_type=F32)


def _rows_grid(n, bk):
    assert n % bk == 0
    return n // bk


def _mm_spec():
    return pl.BlockSpec((C, C), lambda i: (0, 0))


def _bias_spec():
    return pl.BlockSpec((1, C), lambda i: (0, 0))


def tc_data_node(x2, attr8, w1p, b1, w2, b2, wa_enc, wb_dec, b1_dec, v1a, c1):
    n = x2.shape[1]
    v = x2.shape[2]
    bk = 400
    grid = _rows_grid(n, bk)

    def body(x, at, w1, b1r, w2r, b2r, wa, wbd, b1d, va, c1r, o_a, o_bd, o_p):
        xcat = jnp.concatenate([x[0], x[1], at[...]], axis=-1)
        h = jax.nn.silu(_dotb(xcat, w1[...]) + b1r[...])
        xs = _dotb(h, w2r[...]) + b2r[...]
        o_a[...] = _pack2(_dotb(xs, wa[...]))
        o_bd[...] = _pack2(_dotb(xs, wbd[...]) + b1d[...])
        o_p[...] = _dotb(xs, va[...]) + c1r[...]

    row = pl.BlockSpec((bk, C), lambda i: (i, 0))
    half = pl.BlockSpec((bk, H), lambda i: (i, 0))
    return pl.pallas_call(
        body,
        grid=(grid,),
        in_specs=[pl.BlockSpec((2, bk, v), lambda i: (0, i, 0)),
                  pl.BlockSpec((bk, 8), lambda i: (i, 0)),
                  pl.BlockSpec((2 * v + 8, C), lambda i: (0, 0)),
                  _bias_spec(), _mm_spec(), _bias_spec(),
                  _mm_spec(), _mm_spec(), _bias_spec(), _mm_spec(), _bias_spec()],
        out_specs=[half, half, row],
        out_shape=[jax.ShapeDtypeStruct((n, H), jnp.int32),
                   jax.ShapeDtypeStruct((n, H), jnp.int32),
                   jax.ShapeDtypeStruct((n, C), F32)],
    )(x2, attr8, w1p, b1, w2, b2, wa_enc, wb_dec, b1_dec, v1a, c1)


def tc_hidden_node(hp, w1p, b1, w2, b2, wb_enc, b1_enc):
    n = hp.shape[0]
    bk = 400
    grid = _rows_grid(n, bk)

    def body(x, w1, b1r, w2r, b2r, wb, b1e, o_xd, o_b):
        h = jax.nn.silu(_dotb(x[...], w1[...]) + b1r[...])
        xd = _dotb(h, w2r[...]) + b2r[...]
        o_xd[...] = xd
        o_b[...] = _pack2(_dotb(xd, wb[...]) + b1e[...])

    row = pl.BlockSpec((bk, C), lambda i: (i, 0))
    in_row = pl.BlockSpec((bk, 8), lambda i: (i, 0))
    w1_spec = pl.BlockSpec((8, C), lambda i: (0, 0))
    return pl.pallas_call(
        body,
        grid=(grid,),
        in_specs=[in_row, w1_spec, _bias_spec(), _mm_spec(), _bias_spec(),
                  _mm_spec(), _bias_spec()],
        out_specs=[row, pl.BlockSpec((bk, H), lambda i: (i, 0))],
        out_shape=[jax.ShapeDtypeStruct((n, C), F32),
                   jax.ShapeDtypeStruct((10112, H), jnp.int32)],
    )(hp, w1p, b1, w2, b2, wb_enc, b1_enc)


def tc_edge_msg(ua, ub, w2, b2):
    e = ua.shape[0]
    bk = 1024
    grid = _rows_grid(e, bk)

    def body(a, b, w2r, b2r, o):
        u = jax.nn.silu(_unpack2(a[...]) + _unpack2(b[...]))
        m = jnp.dot(u.astype(BF16), w2r[...],
                    preferred_element_type=F32) + b2r[...]
        o[0] = m[:, :H]
        o[1] = m[:, H:]

    half = pl.BlockSpec((bk, H), lambda i: (i, 0))
    return pl.pallas_call(
        body,
        grid=(grid,),
        in_specs=[half, half, _mm_spec(), _bias_spec()],
        out_specs=pl.BlockSpec((2, bk, H), lambda i: (0, i, 0)),
        out_shape=jax.ShapeDtypeStruct((2, e, H), F32),
    )(ua, ub, w2.astype(BF16), b2)


def _ln(y):
    m = jnp.mean(y, axis=-1, keepdims=True)
    v = jnp.mean((y - m) ** 2, axis=-1, keepdims=True)
    return (y - m) * lax.rsqrt(v + 1e-5)


def tc_ln_proj(prev, agg2, wa, wb, b1n):
    n = prev.shape[0]
    bk = 400
    grid = _rows_grid(n, bk)

    def body(p, g, war, wbr, b1r, o_xl, o_a, o_b):
        y = p[...] + jnp.concatenate([g[0], g[1]], axis=-1)
        xl = _ln(y)
        o_xl[...] = xl
        o_a[...] = _pack2(_dotb(xl, war[...]))
        o_b[...] = _pack2(_dotb(xl, wbr[...]) + b1r[...])

    row = pl.BlockSpec((bk, C), lambda i: (i, 0))
    g_spec = pl.BlockSpec((2, bk, H), lambda i: (0, i, 0))
    return pl.pallas_call(
        body,
        grid=(grid,),
        in_specs=[row, g_spec, _mm_spec(), _mm_spec(), _bias_spec()],
        out_specs=[row, pl.BlockSpec((bk, H), lambda i: (i, 0)),
                   pl.BlockSpec((bk, H), lambda i: (i, 0))],
        out_shape=[jax.ShapeDtypeStruct((n, C), F32),
                   jax.ShapeDtypeStruct((10112, H), jnp.int32),
                   jax.ShapeDtypeStruct((10112, H), jnp.int32)],
    )(prev, agg2, wa, wb, b1n)


def tc_dec_src(h1, agg2, xl, wa):
    n = h1.shape[0]
    bk = 400
    grid = _rows_grid(n, bk)

    def body(hh, g, xlr, war, o_a):
        y = hh[...] + jnp.concatenate([g[0], g[1]], axis=-1)
        o_a[...] = _pack2(_dotb(_ln(y) + xlr[...], war[...]))

    row = pl.BlockSpec((bk, C), lambda i: (i, 0))
    g_spec = pl.BlockSpec((2, bk, H), lambda i: (0, i, 0))
    return pl.pallas_call(
        body,
        grid=(grid,),
        in_specs=[row, g_spec, row, _mm_spec()],
        out_specs=pl.BlockSpec((bk, H), lambda i: (i, 0)),
        out_shape=jax.ShapeDtypeStruct((10112, H), jnp.int32),
    )(h1, agg2, xl, wa)


def tc_final(p, agg3, v1b, v2, c2, xlast):
    n = p.shape[0]
    v = v2.shape[1]
    bk = 400
    grid = _rows_grid(n, bk)

    def body(pr, g, vb, v2r, c2r, xl, o):
        gc = jnp.concatenate([g[0], g[1]], axis=-1)
        z = jax.nn.silu(pr[...] + _dotb(gc, vb[...]))
        o[...] = _dotb(z, v2r[...]) + c2r[...] + xl[...]

    row = pl.BlockSpec((bk, C), lambda i: (i, 0))
    g_spec = pl.BlockSpec((2, bk, H), lambda i: (0, i, 0))
    vrow = pl.BlockSpec((bk, v), lambda i: (i, 0))
    return pl.pallas_call(
        body,
        grid=(grid,),
        in_specs=[row, g_spec, _mm_spec(),
                  pl.BlockSpec((C, v), lambda i: (0, 0)),
                  pl.BlockSpec((1, v), lambda i: (0, 0)), vrow],
        out_specs=vrow,
        out_shape=jax.ShapeDtypeStruct((n, v), F32),
    )(p, agg3, v1b, v2, c2, xlast)


def _pad_edges(edge, n_pad, dst_pad_val):
    s = jnp.pad(edge[0], (0, n_pad - edge.shape[1]))
    d = jnp.pad(edge[1], (0, n_pad - edge.shape[1]), constant_values=dst_pad_val)
    return s, d


def kernel(x, params, edge_enc, edge_proc, edge_dec):
    b, t, e_ens, n_data, v = x.shape
    n_hidden = params["hidden_attr"].shape[0]

    x2 = x[0, :, 0]
    attr8 = _pad_cols(params["data_attr"], 8)
    hp = params["hidden_attr"]
    xlast = x[0, -1, 0]

    pe, pd = params["enc_edge"], params["dec_edge"]
    po = params["dec_out"]
    b1e = pe["b1"][None, :]
    b1d = pd["b1"][None, :]
    c1 = po["b1"][None, :]
    v2p = po["W2"]
    c2p = po["b2"][None, :]

    es, ed = params["enc_src"], params["enc_dst"]
    w1_src = _pad_rows(es["W1"], t * v + 8)
    w1_dst = ed["W1"]

    e_enc_p = 155648
    e_proc_p = 163840
    s1, d1 = _pad_edges(edge_enc, e_enc_p, n_hidden)
    s2, d2 = _pad_edges(edge_proc, e_proc_p, n_hidden)
    s3, d3 = _pad_edges(edge_dec, e_enc_p, n_data)

    a_enc, b_dec, p_out = tc_data_node(
        x2, attr8, w1_src, es["b1"][None, :], es["W2"], es["b2"][None, :],
        pe["W1"][:C], pd["W1"][C:], b1d, po["W1"][:C], c1)
    x_dst, b_enc = tc_hidden_node(
        hp, w1_dst, ed["b1"][None, :], ed["W2"], ed["b2"][None, :],
        pe["W1"][C:], b1e)

    seg_h = 10112
    ua, ub = sc_gather2(a_enc, b_enc, s1, d1)
    msg = tc_edge_msg(ua, ub, pe["W2"], pe["b2"][None, :])
    agg = sc_scatter_add(msg, d1, seg_h, 1)

    p0, p1 = params["proc"]
    xl, a0, b0 = tc_ln_proj(x_dst, agg, p0["W1"][:C], p0["W1"][C:],
                            p0["b1"][None, :])
    ua0, ub0 = sc_gather2(a0, b0, s2, d2)
    m0 = tc_edge_msg(ua0, ub0, p0["W2"], p0["b2"][None, :])
    agg0 = sc_scatter_add(m0, d2, seg_h, 1)

    h1, a1, b1 = tc_ln_proj(xl, agg0, p1["W1"][:C], p1["W1"][C:],
                            p1["b1"][None, :])
    ua1, ub1 = sc_gather2(a1, b1, s2, d2)
    m1 = tc_edge_msg(ua1, ub1, p1["W2"], p1["b2"][None, :])
    agg1 = sc_scatter_add(m1, d2, seg_h, 1)

    a_dec = tc_dec_src(h1, agg1, xl, pd["W1"][:C])
    ua3, ub3 = sc_gather2(a_dec, b_dec, s3, d3)
    m3 = tc_edge_msg(ua3, ub3, pd["W2"], pd["b2"][None, :])
    agg3 = sc_scatter_add(m3, d3, 12544, 4)

    o = tc_final(p_out, agg3, po["W1"][C:], v2p, c2p, xlast)
    return o.reshape(b, e_ens, n_data, v)


# speedup vs baseline: 1.1269x; 1.0449x over previous
import functools

import jax
import jax.numpy as jnp
from jax import lax
from jax.experimental import pallas as pl
from jax.experimental.pallas import tpu as pltpu
from jax.experimental.pallas import tpu_sc as plsc

C = 256
H = 128
NCORES = 2
NSUB = 16
NW = NCORES * NSUB
KCH = 128
KCH_S = 64

F32 = jnp.float32
BF16 = jnp.bfloat16


def _pad_rows(a, n):
    return jnp.pad(a, ((0, n - a.shape[0]),) + ((0, 0),) * (a.ndim - 1))


def _pad_cols(a, n):
    return jnp.pad(a, ((0, 0), (0, n - a.shape[1])))


def _pack2(a):
    bl = lax.bitcast_convert_type(a[:, :H], jnp.uint32)
    bh = lax.bitcast_convert_type(a[:, H:], jnp.uint32)
    lo = (bl + jnp.uint32(0x8000)) >> 16
    hi = (bh + jnp.uint32(0x8000)) & jnp.uint32(0xFFFF0000)
    return lax.bitcast_convert_type(hi | lo, jnp.int32)


def _unpack2(u):
    v = lax.bitcast_convert_type(u, jnp.uint32)
    lo = lax.bitcast_convert_type(v << 16, F32)
    hi = lax.bitcast_convert_type(v & jnp.uint32(0xFFFF0000), F32)
    return jnp.concatenate([lo, hi], axis=-1)


def sc_gather2(table_a, table_b, sidx, didx):
    dt = table_a.dtype
    E = sidx.shape[0]
    per_w = E // NW
    nch = per_w // KCH
    assert nch % 2 == 0
    mesh = plsc.VectorSubcoreMesh(core_axis_name="c", subcore_axis_name="s")

    @functools.partial(
        pl.kernel,
        mesh=mesh,
        out_type=(
            jax.ShapeDtypeStruct((E, H), dt),
            jax.ShapeDtypeStruct((E, H), dt),
        ),
        scratch_types=[
            pltpu.VMEM((per_w,), jnp.int32),
            pltpu.VMEM((per_w,), jnp.int32),
            pltpu.VMEM((KCH, H), dt),
            pltpu.VMEM((KCH, H), dt),
            pltpu.VMEM((KCH, H), dt),
            pltpu.VMEM((KCH, H), dt),
            pltpu.SemaphoreType.DMA,
            pltpu.SemaphoreType.DMA,
        ],
    )
    def k(ta, tb, si, di, oa, ob, si_v, di_v, a0, b0, a1, b1, sem0, sem1):
        wid = lax.axis_index("s") * NCORES + lax.axis_index("c")
        base = wid * per_w
        pltpu.sync_copy(si.at[pl.ds(base, per_w)], si_v)
        pltpu.sync_copy(di.at[pl.ds(base, per_w)], di_v)

        def fire(off, ba, bb, sem):
            pltpu.async_copy(ta.at[si_v.at[pl.ds(off, KCH)]], ba, sem)
            pltpu.async_copy(tb.at[di_v.at[pl.ds(off, KCH)]], bb, sem)

        def drain(ba, bb, sem):
            pltpu.make_async_copy(ta.at[pl.ds(0, KCH)], ba, sem).wait()
            pltpu.make_async_copy(tb.at[pl.ds(0, KCH)], bb, sem).wait()

        fire(0, a0, b0, sem0)

        def body(i, carry):
            c0 = 2 * i * KCH
            fire(c0 + KCH, a1, b1, sem1)
            drain(a0, b0, sem0)
            pltpu.sync_copy(a0, oa.at[pl.ds(base + c0, KCH)])
            pltpu.sync_copy(b0, ob.at[pl.ds(base + c0, KCH)])

            @pl.when(i + 1 < nch // 2)
            def _():
                fire(c0 + 2 * KCH, a0, b0, sem0)

            drain(a1, b1, sem1)
            pltpu.sync_copy(a1, oa.at[pl.ds(base + c0 + KCH, KCH)])
            pltpu.sync_copy(b1, ob.at[pl.ds(base + c0 + KCH, KCH)])
            return carry

        lax.fori_loop(0, nch // 2, body, 0)

    return k(table_a, table_b, sidx, didx)


def sc_scatter_add(msg2, didx, seg, rounds):
    E = didx.shape[0]
    per_t = E // NSUB
    nch = per_t // KCH_S
    assert nch % 2 == 0
    zr = seg + 128
    zr_t = zr // NSUB
    cr = seg // NSUB
    nout = rounds * seg
    mesh = plsc.VectorSubcoreMesh(core_axis_name="c", subcore_axis_name="s")

    @functools.partial(
        pl.kernel,
        mesh=mesh,
        out_type=jax.ShapeDtypeStruct((2, nout, H), F32),
        scratch_types=[
            pltpu.VMEM((per_t,), jnp.int32),
            pltpu.VMEM((KCH_S,), jnp.int32),
            pltpu.VMEM((KCH_S,), jnp.int32),
            pltpu.VMEM((KCH_S, H), F32),
            pltpu.VMEM((KCH_S, H), F32),
            pltpu.VMEM_SHARED((zr, H), F32),
            pltpu.SemaphoreType.DMA,
            pltpu.SemaphoreType.DMA,
        ],
    )
    def k(msg, di, zeros_h, out, di_v, idx0, idx1, buf0, buf1, acc, sem0, sem1):
        c = lax.axis_index("c")
        t = lax.axis_index("s")
        base = t * per_t
        pltpu.sync_copy(di.at[pl.ds(base, per_t)], di_v)

        def fire(off, buf, sem):
            pltpu.async_copy(msg.at[c, pl.ds(base + off, KCH_S)], buf, sem)

        def drain(buf, sem):
            pltpu.make_async_copy(msg.at[c, pl.ds(0, KCH_S)], buf, sem).wait()

        def xform(off, idx_v, r):
            for j in range(KCH_S // 16):
                dv = di_v[pl.ds(off + j * 16, 16)]
                lv = dv - r * seg
                lv = jnp.where((lv < 0) | (lv >= seg), seg, lv)
                idx_v[pl.ds(j * 16, 16)] = lv

        for r in range(rounds):
            pltpu.sync_copy(
                zeros_h.at[pl.ds(t * zr_t, zr_t)], acc.at[pl.ds(t * zr_t, zr_t)]
            )
            plsc.subcore_barrier()
            fire(0, buf0, sem0)

            def body(i, carry):
                c0 = 2 * i * KCH_S
                fire(c0 + KCH_S, buf1, sem1)
                xform(c0, idx0, r)
                drain(buf0, sem0)
                pltpu.sync_copy(buf0, acc.at[idx0], add=True)

                @pl.when(i + 1 < nch // 2)
                def _():
                    fire(c0 + 2 * KCH_S, buf0, sem0)

                xform(c0 + KCH_S, idx1, r)
                drain(buf1, sem1)
                pltpu.sync_copy(buf1, acc.at[idx1], add=True)
                return carry

            lax.fori_loop(0, nch // 2, body, 0)
            plsc.subcore_barrier()
            start = t * cr
            pltpu.sync_copy(
                acc.at[pl.ds(start, cr)], out.at[c, pl.ds(r * seg + start, cr)]
            )
            plsc.subcore_barrier()

    zeros_h = jnp.zeros((zr, H), F32)
    return k(msg2, didx, zeros_h)


def _dotb(a, w):
    return jnp.dot(a.astype(BF16), w.astype(BF16), preferred_element_type=F32)


def _rows_grid(n, bk):
    assert n % bk == 0
    return n // bk


def _mm_spec():
    return pl.BlockSpec((C, C), lambda i: (0, 0))


def _bias_spec():
    return pl.BlockSpec((1, C), lambda i: (0, 0))


def tc_data_node(x2, attr8, w1p, b1, w2, b2, wa_enc, wb_dec, b1_dec, v1a, c1):
    n = x2.shape[1]
    v = x2.shape[2]
    bk = 2000
    grid = _rows_grid(n, bk)

    def body(x, at, w1, b1r, w2r, b2r, wa, wbd, b1d, va, c1r, o_a, o_bd, o_p):
        xcat = jnp.concatenate([x[0], x[1], at[...]], axis=-1)
        h = jax.nn.silu(_dotb(xcat, w1[...]) + b1r[...])
        xs = _dotb(h, w2r[...]) + b2r[...]
        o_a[...] = _pack2(_dotb(xs, wa[...]))
        o_bd[...] = _pack2(_dotb(xs, wbd[...]) + b1d[...])
        o_p[...] = _dotb(xs, va[...]) + c1r[...]

    row = pl.BlockSpec((bk, C), lambda i: (i, 0))
    half = pl.BlockSpec((bk, H), lambda i: (i, 0))
    return pl.pallas_call(
        body,
        grid=(grid,),
        in_specs=[pl.BlockSpec((2, bk, v), lambda i: (0, i, 0)),
                  pl.BlockSpec((bk, 8), lambda i: (i, 0)),
                  pl.BlockSpec((2 * v + 8, C), lambda i: (0, 0)),
                  _bias_spec(), _mm_spec(), _bias_spec(),
                  _mm_spec(), _mm_spec(), _bias_spec(), _mm_spec(), _bias_spec()],
        out_specs=[half, half, row],
        out_shape=[jax.ShapeDtypeStruct((n, H), jnp.int32),
                   jax.ShapeDtypeStruct((n, H), jnp.int32),
                   jax.ShapeDtypeStruct((n, C), F32)],
    )(x2, attr8, w1p, b1, w2, b2, wa_enc, wb_dec, b1_dec, v1a, c1)


def tc_hidden_node(hp, w1p, b1, w2, b2, wb_enc, b1_enc):
    n = hp.shape[0]
    bk = 2000
    grid = _rows_grid(n, bk)

    def body(x, w1, b1r, w2r, b2r, wb, b1e, o_xd, o_b):
        h = jax.nn.silu(_dotb(x[...], w1[...]) + b1r[...])
        xd = _dotb(h, w2r[...]) + b2r[...]
        o_xd[...] = xd
        o_b[...] = _pack2(_dotb(xd, wb[...]) + b1e[...])

    row = pl.BlockSpec((bk, C), lambda i: (i, 0))
    in_row = pl.BlockSpec((bk, 8), lambda i: (i, 0))
    w1_spec = pl.BlockSpec((8, C), lambda i: (0, 0))
    return pl.pallas_call(
        body,
        grid=(grid,),
        in_specs=[in_row, w1_spec, _bias_spec(), _mm_spec(), _bias_spec(),
                  _mm_spec(), _bias_spec()],
        out_specs=[row, pl.BlockSpec((bk, H), lambda i: (i, 0))],
        out_shape=[jax.ShapeDtypeStruct((n, C), F32),
                   jax.ShapeDtypeStruct((10112, H), jnp.int32)],
    )(hp, w1p, b1, w2, b2, wb_enc, b1_enc)


def tc_edge_msg(ua, ub, w2, b2):
    e = ua.shape[0]
    bk = 1024
    grid = _rows_grid(e, bk)

    def body(a, b, w2r, b2r, o):
        u = jax.nn.silu(_unpack2(a[...]) + _unpack2(b[...]))
        m = jnp.dot(u.astype(BF16), w2r[...],
                    preferred_element_type=F32) + b2r[...]
        o[0] = m[:, :H]
        o[1] = m[:, H:]

    half = pl.BlockSpec((bk, H), lambda i: (i, 0))
    return pl.pallas_call(
        body,
        grid=(grid,),
        in_specs=[half, half, _mm_spec(), _bias_spec()],
        out_specs=pl.BlockSpec((2, bk, H), lambda i: (0, i, 0)),
        out_shape=jax.ShapeDtypeStruct((2, e, H), F32),
    )(ua, ub, w2.astype(BF16), b2)


def _ln(y):
    m = jnp.mean(y, axis=-1, keepdims=True)
    v = jnp.mean((y - m) ** 2, axis=-1, keepdims=True)
    return (y - m) * lax.rsqrt(v + 1e-5)


def tc_ln_proj(prev, agg2, wa, wb, b1n):
    n = prev.shape[0]
    bk = 2000
    grid = _rows_grid(n, bk)

    def body(p, g, war, wbr, b1r, o_xl, o_a, o_b):
        y = p[...] + jnp.concatenate([g[0], g[1]], axis=-1)
        xl = _ln(y)
        o_xl[...] = xl
        o_a[...] = _pack2(_dotb(xl, war[...]))
        o_b[...] = _pack2(_dotb(xl, wbr[...]) + b1r[...])

    row = pl.BlockSpec((bk, C), lambda i: (i, 0))
    g_spec = pl.BlockSpec((2, bk, H), lambda i: (0, i, 0))
    return pl.pallas_call(
        body,
        grid=(grid,),
        in_specs=[row, g_spec, _mm_spec(), _mm_spec(), _bias_spec()],
        out_specs=[row, pl.BlockSpec((bk, H), lambda i: (i, 0)),
                   pl.BlockSpec((bk, H), lambda i: (i, 0))],
        out_shape=[jax.ShapeDtypeStruct((n, C), F32),
                   jax.ShapeDtypeStruct((10112, H), jnp.int32),
                   jax.ShapeDtypeStruct((10112, H), jnp.int32)],
    )(prev, agg2, wa, wb, b1n)


def tc_dec_src(h1, agg2, xl, wa):
    n = h1.shape[0]
    bk = 2000
    grid = _rows_grid(n, bk)

    def body(hh, g, xlr, war, o_a):
        y = hh[...] + jnp.concatenate([g[0], g[1]], axis=-1)
        o_a[...] = _pack2(_dotb(_ln(y) + xlr[...], war[...]))

    row = pl.BlockSpec((bk, C), lambda i: (i, 0))
    g_spec = pl.BlockSpec((2, bk, H), lambda i: (0, i, 0))
    return pl.pallas_call(
        body,
        grid=(grid,),
        in_specs=[row, g_spec, row, _mm_spec()],
        out_specs=pl.BlockSpec((bk, H), lambda i: (i, 0)),
        out_shape=jax.ShapeDtypeStruct((10112, H), jnp.int32),
    )(h1, agg2, xl, wa)


def tc_final(p, agg3, v1b, v2, c2, xlast):
    n = p.shape[0]
    v = v2.shape[1]
    bk = 2000
    grid = _rows_grid(n, bk)

    def body(pr, g, vb, v2r, c2r, xl, o):
        gc = jnp.concatenate([g[0], g[1]], axis=-1)
        z = jax.nn.silu(pr[...] + _dotb(gc, vb[...]))
        o[...] = _dotb(z, v2r[...]) + c2r[...] + xl[...]

    row = pl.BlockSpec((bk, C), lambda i: (i, 0))
    g_spec = pl.BlockSpec((2, bk, H), lambda i: (0, i, 0))
    vrow = pl.BlockSpec((bk, v), lambda i: (i, 0))
    return pl.pallas_call(
        body,
        grid=(grid,),
        in_specs=[row, g_spec, _mm_spec(),
                  pl.BlockSpec((C, v), lambda i: (0, 0)),
                  pl.BlockSpec((1, v), lambda i: (0, 0)), vrow],
        out_specs=vrow,
        out_shape=jax.ShapeDtypeStruct((n, v), F32),
    )(p, agg3, v1b, v2, c2, xlast)


def _pad_edges(edge, n_pad, dst_pad_val):
    s = jnp.pad(edge[0], (0, n_pad - edge.shape[1]))
    d = jnp.pad(edge[1], (0, n_pad - edge.shape[1]), constant_values=dst_pad_val)
    return s, d


def kernel(x, params, edge_enc, edge_proc, edge_dec):
    b, t, e_ens, n_data, v = x.shape
    n_hidden = params["hidden_attr"].shape[0]

    x2 = x[0, :, 0]
    attr8 = _pad_cols(params["data_attr"], 8)
    hp = params["hidden_attr"]
    xlast = x[0, -1, 0]

    pe, pd = params["enc_edge"], params["dec_edge"]
    po = params["dec_out"]
    b1e = pe["b1"][None, :]
    b1d = pd["b1"][None, :]
    c1 = po["b1"][None, :]
    v2p = po["W2"]
    c2p = po["b2"][None, :]

    es, ed = params["enc_src"], params["enc_dst"]
    w1_src = _pad_rows(es["W1"], t * v + 8)
    w1_dst = ed["W1"]

    e_enc_p = 155648
    e_proc_p = 163840
    s1, d1 = _pad_edges(edge_enc, e_enc_p, n_hidden)
    s2, d2 = _pad_edges(edge_proc, e_proc_p, n_hidden)
    s3, d3 = _pad_edges(edge_dec, e_enc_p, n_data)

    a_enc, b_dec, p_out = tc_data_node(
        x2, attr8, w1_src, es["b1"][None, :], es["W2"], es["b2"][None, :],
        pe["W1"][:C], pd["W1"][C:], b1d, po["W1"][:C], c1)
    x_dst, b_enc = tc_hidden_node(
        hp, w1_dst, ed["b1"][None, :], ed["W2"], ed["b2"][None, :],
        pe["W1"][C:], b1e)

    seg_h = 10112
    ua, ub = sc_gather2(a_enc, b_enc, s1, d1)
    msg = tc_edge_msg(ua, ub, pe["W2"], pe["b2"][None, :])
    agg = sc_scatter_add(msg, d1, seg_h, 1)

    p0, p1 = params["proc"]
    xl, a0, b0 = tc_ln_proj(x_dst, agg, p0["W1"][:C], p0["W1"][C:],
                            p0["b1"][None, :])
    ua0, ub0 = sc_gather2(a0, b0, s2, d2)
    m0 = tc_edge_msg(ua0, ub0, p0["W2"], p0["b2"][None, :])
    agg0 = sc_scatter_add(m0, d2, seg_h, 1)

    h1, a1, b1 = tc_ln_proj(xl, agg0, p1["W1"][:C], p1["W1"][C:],
                            p1["b1"][None, :])
    ua1, ub1 = sc_gather2(a1, b1, s2, d2)
    m1 = tc_edge_msg(ua1, ub1, p1["W2"], p1["b2"][None, :])
    agg1 = sc_scatter_add(m1, d2, seg_h, 1)

    a_dec = tc_dec_src(h1, agg1, xl, pd["W1"][:C])
    ua3, ub3 = sc_gather2(a_dec, b_dec, s3, d3)
    m3 = tc_edge_msg(ua3, ub3, pd["W2"], pd["b2"][None, :])
    agg3 = sc_scatter_add(m3, d3, 12544, 4)

    o = tc_final(p_out, agg3, po["W1"][C:], v2p, c2p, xlast)
    return o.reshape(b, e_ens, n_data, v)
